# Initial kernel scaffold; baseline (speedup 1.0000x reference)
import jax
import jax.numpy as jnp
from jax.experimental import pallas as pl


def kernel(attr_id, attr_category, attr_brand, attr_shop, W_id, W_category, W_brand, W_shop):
    raise NotImplementedError("write your pallas kernel here")


# speedup vs baseline: 4.9028x; 4.9028x over previous
import jax
import jax.numpy as jnp
from jax import lax
from jax.experimental import pallas as pl
from jax.experimental.pallas import tpu as pltpu
from jax.experimental.pallas import tpu_sc as plsc

B, L = 4096, 50
N = B * L
D_OUT = 256
NC, NS = 2, 16
NW = NC * NS
PER_W = N // NW
C = 256
NCHUNK = PER_W // C
G16 = C // 16


def _body(i0, i1, i2, i3, w_id, w_cat, w_br, w_sh, out_hbm,
          iid, icat, ibr, ish, bid, bR, tmp, s0, s1, s2):
    wid = lax.axis_index("s") * NC + lax.axis_index("c")
    w_base = wid * PER_W

    def chunk(g, carry):
        base = w_base + g * C
        pltpu.sync_copy(i0.at[pl.ds(base, C)], iid)
        pltpu.sync_copy(i1.at[pl.ds(base, C)], icat)
        pltpu.sync_copy(i2.at[pl.ds(base, C)], ibr)
        pltpu.sync_copy(i3.at[pl.ds(base, C)], ish)
        d0 = pltpu.async_copy(w_id.at[iid], bid, s0)
        d1 = pltpu.async_copy(w_cat.at[icat], bR, s1)
        d2 = pltpu.async_copy(w_br.at[ibr], tmp, s2)
        d0.wait()
        d1.wait()
        d2.wait()

        def row_br(r, c2):
            for j in range(4):
                bR[r, pl.ds(32 + j * 16, 16)] = tmp[r, pl.ds(j * 16, 16)]
            return c2
        lax.fori_loop(0, C, row_br, 0)

        pltpu.async_copy(w_sh.at[ish], tmp, s2).wait()

        def row_sh(r, c2):
            for j in range(2):
                bR[r, pl.ds(96 + j * 16, 16)] = tmp[r, pl.ds(j * 16, 16)]
            return c2
        lax.fori_loop(0, C, row_sh, 0)

        acc = jnp.zeros((16,), jnp.int32)
        for gg in range(G16):
            iv = iid[pl.ds(gg * 16, 16)]
            acc = acc | jnp.where(iv == 0, 1, 0)
        nz = jnp.max(acc)

        @pl.when(nz > 0)
        def _fix():
            def per_group(i, c2):
                iv = iid[pl.ds(i * 16, 16)]
                z = iv == 0
                rows = lax.iota(jnp.int32, 16) + i * 16
                zf = jnp.zeros((16,), jnp.float32)
                for col in range(128):
                    cols = jnp.full((16,), col, jnp.int32)
                    plsc.store_scatter(bid, [rows, cols], zf, mask=z)
                return c2
            lax.fori_loop(0, G16, per_group, 0)

        pltpu.sync_copy(bid, out_hbm.at[pl.ds(base, C), pl.ds(0, 128)])
        pltpu.sync_copy(bR, out_hbm.at[pl.ds(base, C), pl.ds(128, 128)])
        return carry

    lax.fori_loop(0, NCHUNK, chunk, 0)


def kernel(attr_id, attr_category, attr_brand, attr_shop,
           W_id, W_category, W_brand, W_shop):
    ii = attr_id.astype(jnp.int32).reshape(N)
    ic = attr_category.astype(jnp.int32).reshape(N)
    ib = attr_brand.astype(jnp.int32).reshape(N)
    ish = attr_shop.astype(jnp.int32).reshape(N)
    w_cat = jnp.pad(W_category, ((0, 0), (0, 96)))
    w_br = jnp.pad(W_brand, ((0, 0), (0, 64)))
    w_sh = jnp.pad(W_shop, ((0, 0), (0, 96)))
    k = pl.kernel(
        _body,
        out_type=jax.ShapeDtypeStruct((N, D_OUT), jnp.float32),
        mesh=plsc.VectorSubcoreMesh(core_axis_name="c", subcore_axis_name="s"),
        compiler_params=pltpu.CompilerParams(needs_layout_passes=False),
        scratch_types=[
            pltpu.VMEM((C,), jnp.int32),
            pltpu.VMEM((C,), jnp.int32),
            pltpu.VMEM((C,), jnp.int32),
            pltpu.VMEM((C,), jnp.int32),
            pltpu.VMEM((C, 128), jnp.float32),
            pltpu.VMEM((C, 128), jnp.float32),
            pltpu.VMEM((C, 128), jnp.float32),
            pltpu.SemaphoreType.DMA,
            pltpu.SemaphoreType.DMA,
            pltpu.SemaphoreType.DMA,
        ],
    )
    out = k(ii, ic, ib, ish, W_id, w_cat, w_br, w_sh)
    return out.reshape(B, L, D_OUT)


# speedup vs baseline: 5.8444x; 1.1920x over previous
import jax
import jax.numpy as jnp
from jax import lax
from jax.experimental import pallas as pl
from jax.experimental.pallas import tpu as pltpu
from jax.experimental.pallas import tpu_sc as plsc

B, L = 4096, 50
N = B * L
D_OUT = 256
NC, NS = 2, 16
NW = NC * NS
PER_W = N // NW
C = 80
NCHUNK = PER_W // C
NPAIR = NCHUNK // 2
G16 = C // 16


def _body(i0, i1, i2, i3, w_id, w_cat, w_br, w_sh, out_hbm,
          jid, jcat, jbr, jsh,
          bid0, bR0, tmpB0, tmpS0, bid1, bR1, tmpB1, tmpS1,
          gs0, gs1, ws0, ws1):
    wid = lax.axis_index("s") * NC + lax.axis_index("c")
    w_base = wid * PER_W

    pltpu.sync_copy(i0.at[pl.ds(w_base, PER_W)], jid)
    pltpu.sync_copy(i1.at[pl.ds(w_base, PER_W)], jcat)
    pltpu.sync_copy(i2.at[pl.ds(w_base, PER_W)], jbr)
    pltpu.sync_copy(i3.at[pl.ds(w_base, PER_W)], jsh)

    side = [(bid0, bR0, tmpB0, tmpS0, gs0, ws0),
            (bid1, bR1, tmpB1, tmpS1, gs1, ws1)]

    def fire_gathers(g, s):
        bid, bR, tmpB, tmpS, gs, _ = side[s]
        off = g * C
        pltpu.async_copy(w_id.at[jid.at[pl.ds(off, C)]], bid, gs)
        pltpu.async_copy(w_cat.at[jcat.at[pl.ds(off, C)]], bR, gs)
        pltpu.async_copy(w_br.at[jbr.at[pl.ds(off, C)]], tmpB, gs)
        pltpu.async_copy(w_sh.at[jsh.at[pl.ds(off, C)]], tmpS, gs)

    def drain_gathers(g, s):
        bid, bR, tmpB, tmpS, gs, _ = side[s]
        off = g * C
        pltpu.make_async_copy(w_id.at[jid.at[pl.ds(off, C)]], bid, gs).wait()
        pltpu.make_async_copy(w_cat.at[jcat.at[pl.ds(off, C)]], bR, gs).wait()
        pltpu.make_async_copy(w_br.at[jbr.at[pl.ds(off, C)]], tmpB, gs).wait()
        pltpu.make_async_copy(w_sh.at[jsh.at[pl.ds(off, C)]], tmpS, gs).wait()

    def assemble_fix(g, s):
        bid, bR, tmpB, tmpS, _, _ = side[s]

        def row(r, c2):
            for j in range(4):
                bR[r, pl.ds(32 + j * 16, 16)] = tmpB[r, pl.ds(j * 16, 16)]
            for j in range(2):
                bR[r, pl.ds(96 + j * 16, 16)] = tmpS[r, pl.ds(j * 16, 16)]
            return c2
        lax.fori_loop(0, C, row, 0)

        off = g * C
        acc = jnp.zeros((16,), jnp.int32)
        for gg in range(G16):
            iv = jid[pl.ds(off + gg * 16, 16)]
            acc = acc | jnp.where(iv == 0, 1, 0)
        nz = jnp.max(acc)

        @pl.when(nz > 0)
        def _fix():
            def per_group(i, c2):
                iv = jid[pl.ds(off + i * 16, 16)]
                z = iv == 0
                rows = lax.iota(jnp.int32, 16) + i * 16
                zf = jnp.zeros((16,), jnp.float32)
                for col in range(128):
                    cols = jnp.full((16,), col, jnp.int32)
                    plsc.store_scatter(bid, [rows, cols], zf, mask=z)
                return c2
            lax.fori_loop(0, G16, per_group, 0)

    def fire_writes(g, s):
        bid, bR, _, _, _, ws = side[s]
        base = w_base + g * C
        pltpu.async_copy(bid, out_hbm.at[pl.ds(base, C), pl.ds(0, 128)], ws)
        pltpu.async_copy(bR, out_hbm.at[pl.ds(base, C), pl.ds(128, 128)], ws)

    def drain_writes(g, s):
        bid, bR, _, _, _, ws = side[s]
        base = w_base + g * C
        pltpu.make_async_copy(
            bid, out_hbm.at[pl.ds(base, C), pl.ds(0, 128)], ws).wait()
        pltpu.make_async_copy(
            bR, out_hbm.at[pl.ds(base, C), pl.ds(128, 128)], ws).wait()

    fire_gathers(0, 0)
    fire_gathers(1, 1)

    def pair(i, carry):
        a = 2 * i
        b = a + 1
        drain_gathers(a, 0)
        assemble_fix(a, 0)
        fire_writes(a, 0)
        drain_gathers(b, 1)
        assemble_fix(b, 1)
        fire_writes(b, 1)
        drain_writes(a, 0)

        @pl.when(i < NPAIR - 1)
        def _n0():
            fire_gathers(a + 2, 0)
        drain_writes(b, 1)

        @pl.when(i < NPAIR - 1)
        def _n1():
            fire_gathers(b + 2, 1)
        return carry

    lax.fori_loop(0, NPAIR, pair, 0)


def kernel(attr_id, attr_category, attr_brand, attr_shop,
           W_id, W_category, W_brand, W_shop):
    ii = attr_id.astype(jnp.int32).reshape(N)
    ic = attr_category.astype(jnp.int32).reshape(N)
    ib = attr_brand.astype(jnp.int32).reshape(N)
    ish = attr_shop.astype(jnp.int32).reshape(N)
    w_cat = jnp.pad(W_category, ((0, 0), (0, 96)))
    w_br = jnp.pad(W_brand, ((0, 0), (0, 64)))
    w_sh = jnp.pad(W_shop, ((0, 0), (0, 96)))
    k = pl.kernel(
        _body,
        out_type=jax.ShapeDtypeStruct((N, D_OUT), jnp.float32),
        mesh=plsc.VectorSubcoreMesh(core_axis_name="c", subcore_axis_name="s"),
        compiler_params=pltpu.CompilerParams(needs_layout_passes=False),
        scratch_types=[
            pltpu.VMEM((PER_W,), jnp.int32),
            pltpu.VMEM((PER_W,), jnp.int32),
            pltpu.VMEM((PER_W,), jnp.int32),
            pltpu.VMEM((PER_W,), jnp.int32),
            pltpu.VMEM((C, 128), jnp.float32),
            pltpu.VMEM((C, 128), jnp.float32),
            pltpu.VMEM((C, 128), jnp.float32),
            pltpu.VMEM((C, 128), jnp.float32),
            pltpu.VMEM((C, 128), jnp.float32),
            pltpu.VMEM((C, 128), jnp.float32),
            pltpu.VMEM((C, 128), jnp.float32),
            pltpu.VMEM((C, 128), jnp.float32),
            pltpu.SemaphoreType.DMA,
            pltpu.SemaphoreType.DMA,
            pltpu.SemaphoreType.DMA,
            pltpu.SemaphoreType.DMA,
        ],
    )
    out = k(ii, ic, ib, ish, W_id, w_cat, w_br, w_sh)
    return out.reshape(B, L, D_OUT)
